# baseline (speedup 1.0000x reference)
import functools

import jax
import jax.numpy as jnp
from jax import lax
from jax.experimental import pallas as pl
from jax.experimental.pallas import tpu as pltpu
from jax.experimental.pallas import tpu_sc as plsc

N = 320000
D = 128
G = 1024

_NS = 16
_L = 16
CHUNK = 128
NCHUNKS = N // CHUNK
CPT = NCHUNKS // _NS
NEXTRA = NCHUNKS - CPT * _NS
DPT = CPT // 2
NBUF = 2
AHEAD = 1
NSEXT = DPT // NBUF
IDXROWS = CPT + 12
NCPAD = 2512
GSTRIPE = G // _NS


def _sc_segment_sums(x_paper, x_author, batch_paper, batch_author):
    mesh = plsc.VectorSubcoreMesh(core_axis_name="c", subcore_axis_name="s")

    @functools.partial(
        pl.kernel,
        mesh=mesh,
        out_type=[
            jax.ShapeDtypeStruct((2 * G, D), jnp.float32),
            jax.ShapeDtypeStruct((2, _NS, G), jnp.float32),
        ],
        scratch_types=(
            [pltpu.VMEM((2 * CHUNK, D), jnp.float32)] * NBUF
            + [
                pltpu.VMEM((IDXROWS, CHUNK), jnp.int32),
                pltpu.VMEM((G,), jnp.float32),
                pltpu.VMEM((_L, D), jnp.float32),
                pltpu.VMEM((1, _L), jnp.int32),
                pltpu.VMEM_SHARED((G, D), jnp.float32),
            ]
            + [pltpu.SemaphoreType.DMA] * NBUF
            + [pltpu.SemaphoreType.DMA] * (2 * NBUF)
            + [pltpu.SemaphoreType.DMA]
        ),
        compiler_params=pltpu.CompilerParams(needs_layout_passes=False),
    )
    def k(xp, xa, bp2d, ba2d, zeros_gd, sums_out, cnts_out, *refs):
        rows = refs[:NBUF]
        idx_blk, hist, flush_buf, seg_row, acc = refs[NBUF:NBUF + 5]
        ls = refs[NBUF + 5:2 * NBUF + 5]
        ss = refs[2 * NBUF + 5:4 * NBUF + 5]
        fs = refs[4 * NBUF + 5]
        cid = lax.axis_index("c")
        sid = lax.axis_index("s")

        pltpu.sync_copy(zeros_gd.at[pl.ds(sid * GSTRIPE, GSTRIPE)],
                        acc.at[pl.ds(sid * GSTRIPE, GSTRIPE)])
        zero16 = jnp.zeros((_L,), jnp.float32)
        for i in range(G // _L):
            hist[pl.ds(i * _L, _L)] = zero16
        plsc.subcore_barrier()

        one16 = jnp.ones((_L,), jnp.float32)

        def process(x_hbm, b2d):
            extra_before = jnp.maximum(sid - (_NS - NEXTRA), 0)
            chunk0 = sid * CPT + extra_before
            row_base = chunk0 * CHUNK
            a0 = (chunk0 // 8) * 8
            doff = chunk0 - a0

            def load(d_local, buf, sem):
                pltpu.async_copy(
                    x_hbm.at[pl.ds(row_base + d_local * 2 * CHUNK, 2 * CHUNK)],
                    buf, sem)

            def scat(c_local, buf, sub, sem):
                pltpu.async_copy(buf.at[pl.ds(sub * CHUNK, CHUNK)],
                                 acc.at[idx_blk.at[doff + c_local]], sem,
                                 add=True)

            def wait_load(buf, sem):
                pltpu.make_async_copy(x_hbm.at[pl.ds(0, 2 * CHUNK)], buf,
                                      sem).wait()

            def wait_scat(buf, sem):
                pltpu.make_async_copy(buf.at[pl.ds(0, CHUNK)],
                                      acc.at[idx_blk.at[0]], sem).wait()

            def count(c_local):
                for j in range(CHUNK // _L):
                    idx16 = idx_blk[doff + c_local, pl.ds(j * _L, _L)]
                    plsc.addupdate_scatter(hist, [idx16], one16)

            def wait_flush():
                pltpu.make_async_copy(flush_buf, acc.at[seg_row.at[0]],
                                      fs).wait()

            def single_flush(b, sub, first, ff):
                def tbody(r, sacc):
                    out = []
                    for j in range(D // _L):
                        s = sacc[j]
                        for rr in range(_L):
                            s = s + rows[b][sub * CHUNK + r * _L + rr,
                                            pl.ds(j * _L, _L)]
                        out.append(s)
                    return tuple(out)

                zero = jnp.zeros((_L,), jnp.float32)
                sums8 = lax.fori_loop(0, CHUNK // _L, tbody, (zero,) * (D // _L))
                pl.when(ff > 0)(wait_flush)
                seg_row[0, pl.ds(0, _L)] = jnp.full((_L,), first, jnp.int32)
                for j in range(D // _L):
                    flush_buf[0, pl.ds(j * _L, _L)] = sums8[j]
                pltpu.async_copy(flush_buf, acc.at[seg_row.at[0]], fs, add=True)

            for b in range(AHEAD):
                load(b, rows[b], ls[b])
            pltpu.sync_copy(b2d.at[pl.ds(a0, IDXROWS)], idx_blk)
            pltpu.sync_copy(zeros_gd.at[pl.ds(0, _L)], flush_buf)

            def body(q, carry):
                ff = carry[0]
                fl = list(carry[1:])
                for b in range(NBUF):
                    d = NBUF * q + b
                    wait_load(rows[b], ls[b])
                    for sub in range(2):
                        c = 2 * d + sub
                        dc = doff + c
                        vf = idx_blk[dc, pl.ds(0, _L)]
                        vl = idx_blk[dc, pl.ds(CHUNK - _L, _L)]
                        first = jnp.min(vf)
                        issingle = first == jnp.max(vl)
                        count(c)
                        pl.when(jnp.logical_not(issingle))(
                            lambda b=b, c=c, sub=sub: scat(
                                c, rows[b], sub, ss[2 * b + sub]))
                        pl.when(issingle)(
                            lambda b=b, sub=sub, first=first, ff=ff:
                                single_flush(b, sub, first, ff))
                        ff = jnp.where(issingle, 1, ff)
                        fl[2 * b + sub] = jnp.where(issingle, 0, 1)
                    t = (b + AHEAD) % NBUF
                    for sub in range(2):
                        pl.when(fl[2 * t + sub] > 0)(
                            lambda t=t, sub=sub: wait_scat(
                                rows[t], ss[2 * t + sub]))
                        fl[2 * t + sub] = jnp.int32(0)
                    qmax = (DPT - 1 - AHEAD - b) // NBUF
                    if qmax >= NSEXT - 1:
                        load(d + AHEAD, rows[t], ls[t])
                    else:
                        pl.when(q <= qmax)(
                            lambda d=d, t=t: load(d + AHEAD, rows[t], ls[t]))
                return (ff, *fl)

            zero_flags = (jnp.int32(0),) * (2 * NBUF + 1)
            final = lax.fori_loop(0, NSEXT, body, zero_flags)
            for b in range(NBUF):
                for sub in range(2):
                    pl.when(final[1 + 2 * b + sub] > 0)(
                        lambda b=b, sub=sub: wait_scat(rows[b],
                                                       ss[2 * b + sub]))
            pl.when(final[0] > 0)(wait_flush)

            @pl.when(sid >= _NS - NEXTRA)
            def _():
                pltpu.sync_copy(x_hbm.at[pl.ds(row_base + CPT * CHUNK, CHUNK)],
                                rows[0].at[pl.ds(0, CHUNK)])
                pltpu.sync_copy(rows[0].at[pl.ds(0, CHUNK)],
                                acc.at[idx_blk.at[doff + CPT]], add=True)
                count(CPT)

        pl.when(cid == 0)(lambda: process(xp, bp2d))
        pl.when(cid == 1)(lambda: process(xa, ba2d))

        pltpu.sync_copy(hist, cnts_out.at[cid, sid])

        plsc.subcore_barrier()
        off = cid * G + sid * GSTRIPE
        pltpu.sync_copy(acc.at[pl.ds(sid * GSTRIPE, GSTRIPE)],
                        sums_out.at[pl.ds(off, GSTRIPE)])

    def _b2d(b):
        b2 = b.reshape(NCHUNKS, CHUNK)
        return jnp.concatenate(
            [b2, jnp.zeros((NCPAD - NCHUNKS, CHUNK), jnp.int32)], axis=0)

    zeros_gd = jnp.zeros((G, D), jnp.float32)
    return k(x_paper, x_author, _b2d(batch_paper), _b2d(batch_author), zeros_gd)


def _finalize(sums, cnts):
    def body(s_ref, c_ref, o_ref):
        cp = jnp.maximum(jnp.sum(c_ref[0], axis=0), 1.0).reshape(G, 1)
        ca = jnp.maximum(jnp.sum(c_ref[1], axis=0), 1.0).reshape(G, 1)
        o_ref[...] = s_ref[:G] / cp + s_ref[G:] / ca

    return pl.pallas_call(
        body,
        out_shape=jax.ShapeDtypeStruct((G, D), jnp.float32),
    )(sums, cnts)


def kernel(x_paper, x_author, batch_paper, batch_author):
    sums, cnts = _sc_segment_sums(x_paper, x_author, batch_paper, batch_author)
    return _finalize(sums, cnts)


# speedup vs baseline: 1.1844x; 1.1844x over previous
import functools

import jax
import jax.numpy as jnp
from jax import lax
from jax.experimental import pallas as pl
from jax.experimental.pallas import tpu as pltpu
from jax.experimental.pallas import tpu_sc as plsc

N = 320000
D = 128
G = 1024

_NS = 16
_L = 16
CHUNK = 128
NCHUNKS = N // CHUNK
CPT = NCHUNKS // _NS
NEXTRA = NCHUNKS - CPT * _NS
NBUF = 4
AHEAD = 2
NSEXT = CPT // NBUF
IDXROWS = CPT + 12
NCPAD = 2512
GSTRIPE = G // _NS


def _sc_segment_sums(x_paper, x_author, batch_paper, batch_author):
    mesh = plsc.VectorSubcoreMesh(core_axis_name="c", subcore_axis_name="s")

    @functools.partial(
        pl.kernel,
        mesh=mesh,
        out_type=[
            jax.ShapeDtypeStruct((2 * G, D), jnp.float32),
            jax.ShapeDtypeStruct((2, _NS, G), jnp.float32),
        ],
        scratch_types=(
            [pltpu.VMEM((CHUNK, D), jnp.float32)] * NBUF
            + [
                pltpu.VMEM((IDXROWS, CHUNK), jnp.int32),
                pltpu.VMEM((G,), jnp.float32),
                pltpu.VMEM((_L, D), jnp.float32),
                pltpu.VMEM((1, _L), jnp.int32),
                pltpu.VMEM_SHARED((G, D), jnp.float32),
            ]
            + [pltpu.SemaphoreType.DMA] * NBUF
            + [pltpu.SemaphoreType.DMA] * NBUF
            + [pltpu.SemaphoreType.DMA]
        ),
        compiler_params=pltpu.CompilerParams(needs_layout_passes=False),
    )
    def k(xp, xa, bp2d, ba2d, zeros_gd, sums_out, cnts_out, *refs):
        rows = refs[:NBUF]
        idx_blk, hist, flush_buf, seg_row, acc = refs[NBUF:NBUF + 5]
        ls = refs[NBUF + 5:2 * NBUF + 5]
        ss = refs[2 * NBUF + 5:3 * NBUF + 5]
        fs = refs[3 * NBUF + 5]
        cid = lax.axis_index("c")
        sid = lax.axis_index("s")

        pltpu.sync_copy(zeros_gd.at[pl.ds(sid * GSTRIPE, GSTRIPE)],
                        acc.at[pl.ds(sid * GSTRIPE, GSTRIPE)])
        zero16 = jnp.zeros((_L,), jnp.float32)
        for i in range(G // _L):
            hist[pl.ds(i * _L, _L)] = zero16
        plsc.subcore_barrier()

        one16 = jnp.ones((_L,), jnp.float32)

        def process(x_hbm, b2d):
            extra_before = jnp.maximum(sid - (_NS - NEXTRA), 0)
            chunk0 = sid * CPT + extra_before
            row_base = chunk0 * CHUNK
            a0 = (chunk0 // 8) * 8
            doff = chunk0 - a0

            def load(c_local, buf, sem):
                pltpu.async_copy(
                    x_hbm.at[pl.ds(row_base + c_local * CHUNK, CHUNK)], buf, sem)

            def scat(c_local, buf, sem):
                pltpu.async_copy(buf, acc.at[idx_blk.at[doff + c_local]], sem,
                                 add=True)

            def wait_load(buf, sem):
                pltpu.make_async_copy(x_hbm.at[pl.ds(0, CHUNK)], buf, sem).wait()

            def wait_scat(buf, sem):
                pltpu.make_async_copy(buf, acc.at[idx_blk.at[0]], sem).wait()

            def count(c_local):
                for j in range(CHUNK // _L):
                    idx16 = idx_blk[doff + c_local, pl.ds(j * _L, _L)]
                    plsc.addupdate_scatter(hist, [idx16], one16)

            def wait_flush():
                pltpu.make_async_copy(flush_buf, acc.at[seg_row.at[0]],
                                      fs).wait()

            def single_flush(b, first, ff):
                def tbody(r, sacc):
                    out = []
                    for j in range(D // _L):
                        s = sacc[j]
                        for rr in range(_L):
                            s = s + rows[b][r * _L + rr, pl.ds(j * _L, _L)]
                        out.append(s)
                    return tuple(out)

                zero = jnp.zeros((_L,), jnp.float32)
                sums8 = lax.fori_loop(0, CHUNK // _L, tbody, (zero,) * (D // _L))
                pl.when(ff > 0)(wait_flush)
                seg_row[0, pl.ds(0, _L)] = jnp.full((_L,), first, jnp.int32)
                for j in range(D // _L):
                    flush_buf[0, pl.ds(j * _L, _L)] = sums8[j]
                pltpu.async_copy(flush_buf, acc.at[seg_row.at[0]], fs, add=True)

            for b in range(AHEAD):
                load(b, rows[b], ls[b])
            pltpu.sync_copy(b2d.at[pl.ds(a0, IDXROWS)], idx_blk)
            pltpu.sync_copy(zeros_gd.at[pl.ds(0, _L)], flush_buf)

            def body(q, carry):
                ff = carry[0]
                fl = list(carry[1:])
                for b in range(NBUF):
                    c = NBUF * q + b
                    dc = doff + c
                    wait_load(rows[b], ls[b])
                    vf = idx_blk[dc, pl.ds(0, _L)]
                    vl = idx_blk[dc, pl.ds(CHUNK - _L, _L)]
                    first = jnp.min(vf)
                    issingle = first == jnp.max(vl)
                    count(c)
                    pl.when(jnp.logical_not(issingle))(
                        lambda b=b, c=c: scat(c, rows[b], ss[b]))
                    pl.when(issingle)(
                        lambda b=b, first=first, ff=ff: single_flush(
                            b, first, ff))
                    ff = jnp.where(issingle, 1, ff)
                    fl[b] = jnp.where(issingle, 0, 1)
                    t = (b + AHEAD) % NBUF
                    pl.when(fl[t] > 0)(lambda t=t: wait_scat(rows[t], ss[t]))
                    fl[t] = jnp.int32(0)
                    qmax = (CPT - 1 - AHEAD - b) // NBUF
                    if qmax >= NSEXT - 1:
                        load(c + AHEAD, rows[t], ls[t])
                    else:
                        pl.when(q <= qmax)(
                            lambda c=c, t=t: load(c + AHEAD, rows[t], ls[t]))
                return (ff, *fl)

            zero_flags = (jnp.int32(0),) * (NBUF + 1)
            final = lax.fori_loop(0, NSEXT, body, zero_flags)
            for b in range(NBUF):
                pl.when(final[1 + b] > 0)(
                    lambda b=b: wait_scat(rows[b], ss[b]))
            pl.when(final[0] > 0)(wait_flush)

            @pl.when(sid >= _NS - NEXTRA)
            def _():
                pltpu.sync_copy(x_hbm.at[pl.ds(row_base + CPT * CHUNK, CHUNK)],
                                rows[0])
                pltpu.sync_copy(rows[0], acc.at[idx_blk.at[doff + CPT]],
                                add=True)
                count(CPT)

        pl.when(cid == 0)(lambda: process(xp, bp2d))
        pl.when(cid == 1)(lambda: process(xa, ba2d))

        pltpu.sync_copy(hist, cnts_out.at[cid, sid])

        plsc.subcore_barrier()
        off = cid * G + sid * GSTRIPE
        pltpu.sync_copy(acc.at[pl.ds(sid * GSTRIPE, GSTRIPE)],
                        sums_out.at[pl.ds(off, GSTRIPE)])

    def _b2d(b):
        b2 = b.reshape(NCHUNKS, CHUNK)
        return jnp.concatenate(
            [b2, jnp.zeros((NCPAD - NCHUNKS, CHUNK), jnp.int32)], axis=0)

    zeros_gd = jnp.zeros((G, D), jnp.float32)
    return k(x_paper, x_author, _b2d(batch_paper), _b2d(batch_author), zeros_gd)


def _finalize(sums, cnts):
    def body(s_ref, c_ref, o_ref):
        cp = jnp.maximum(jnp.sum(c_ref[0], axis=0), 1.0).reshape(G, 1)
        ca = jnp.maximum(jnp.sum(c_ref[1], axis=0), 1.0).reshape(G, 1)
        o_ref[...] = s_ref[:G] / cp + s_ref[G:] / ca

    return pl.pallas_call(
        body,
        out_shape=jax.ShapeDtypeStruct((G, D), jnp.float32),
    )(sums, cnts)


def kernel(x_paper, x_author, batch_paper, batch_author):
    sums, cnts = _sc_segment_sums(x_paper, x_author, batch_paper, batch_author)
    return _finalize(sums, cnts)
